# baseline (speedup 1.0000x reference)
import jax
import jax.numpy as jnp
from jax.experimental import pallas as pl
from jax.experimental.pallas import tpu as pltpu


def _pos_kernel(row_ref, col_ref, out_ref, pat, sems):
    h, d = row_ref.shape
    w = col_ref.shape[0]
    b = out_ref.shape[0]
    col_t = col_ref[...].T
    row_t = row_ref[...].T
    pat[:d] = jnp.broadcast_to(col_t[:, None, :], (d, h, w))
    pat[d:] = jnp.broadcast_to(row_t[:, :, None], (d, h, w))
    copies = [
        pltpu.make_async_copy(pat, out_ref.at[i], sems.at[i])
        for i in range(b)
    ]
    for i, c in enumerate(copies):
        c.start(priority=i % 2)
    for c in copies:
        c.wait()


def kernel(pixel_values, row_embedding, column_embedding):
    b = pixel_values.shape[0]
    h, w = pixel_values.shape[-2], pixel_values.shape[-1]
    d = row_embedding.shape[-1]
    row = row_embedding[:h]
    col = column_embedding[:w]
    return pl.pallas_call(
        _pos_kernel,
        in_specs=[
            pl.BlockSpec((h, d), lambda: (0, 0)),
            pl.BlockSpec((w, d), lambda: (0, 0)),
        ],
        out_specs=pl.BlockSpec(memory_space=pl.ANY),
        out_shape=jax.ShapeDtypeStruct((b, 2 * d, h, w), jnp.float32),
        scratch_shapes=[
            pltpu.VMEM((2 * d, h, w), jnp.float32),
            pltpu.SemaphoreType.DMA((b,)),
        ],
    )(row, col)


# speedup vs baseline: 7.1370x; 7.1370x over previous
import jax
import jax.numpy as jnp
from jax.experimental import pallas as pl
from jax.experimental.pallas import tpu as pltpu


def _pos_kernel(row_ref, col_ref, out_ref, pat, sems):
    h, d = row_ref.shape
    w = col_ref.shape[0]
    b = out_ref.shape[0]
    pat[:, :, :d] = jnp.broadcast_to(col_ref[...][None, :, :], (h, w, d))
    pat[:, :, d:] = jnp.broadcast_to(row_ref[...][:, None, :], (h, w, d))
    copies = [
        pltpu.make_async_copy(pat, out_ref.at[i], sems.at[i])
        for i in range(b)
    ]
    for i, c in enumerate(copies):
        c.start(priority=i % 2)
    for c in copies:
        c.wait()


def kernel(pixel_values, row_embedding, column_embedding):
    b = pixel_values.shape[0]
    h, w = pixel_values.shape[-2], pixel_values.shape[-1]
    d = row_embedding.shape[-1]
    row = row_embedding[:h]
    col = column_embedding[:w]
    out = pl.pallas_call(
        _pos_kernel,
        in_specs=[
            pl.BlockSpec((h, d), lambda: (0, 0)),
            pl.BlockSpec((w, d), lambda: (0, 0)),
        ],
        out_specs=pl.BlockSpec(memory_space=pl.ANY),
        out_shape=jax.ShapeDtypeStruct((b, h, w, 2 * d), jnp.float32),
        scratch_shapes=[
            pltpu.VMEM((h, w, 2 * d), jnp.float32),
            pltpu.SemaphoreType.DMA((b,)),
        ],
    )(row, col)
    return jnp.transpose(out, (0, 3, 1, 2))
